# baseline (speedup 1.0000x reference)
import functools

import jax
import jax.numpy as jnp
from jax import lax
from jax.experimental import pallas as pl
from jax.experimental.pallas import tpu as pltpu
from jax.experimental.pallas import tpu_sc as plsc

N = 10000
D = 128
NPAD = 10240
ZR = NPAD // 16
E = 320000
NW = 32
CH = 128
E_NCH = 80
EPAD = NW * E_NCH * CH
L = 100000
L_NCH = 25
LPAD = NW * L_NCH * CH
BM = 1024

E_NCH_C = (158, 2)
LCH = 64
L_TNCH = 50


def _mesh():
    return plsc.VectorSubcoreMesh(core_axis_name="c", subcore_axis_name="s")


def _sc_degree(dst_idx, zeros1):
    @functools.partial(
        pl.kernel,
        out_type=jax.ShapeDtypeStruct((2, NPAD), jnp.float32),
        mesh=_mesh(),
        scratch_types=[
            pltpu.VMEM((E_NCH, CH), jnp.int32),
            pltpu.VMEM((CH,), jnp.float32),
            pltpu.VMEM_SHARED((NPAD,), jnp.float32),
            pltpu.SemaphoreType.DMA,
        ],
    )
    def deg_kernel(dst_hbm, z_hbm, out_hbm, idx_v, ones_v, deg_sh, sem):
        c = lax.axis_index("c")
        s = lax.axis_index("s")
        wid = c * 16 + s
        pltpu.sync_copy(z_hbm.at[pl.ds(s * ZR, ZR)], deg_sh.at[pl.ds(s * ZR, ZR)])
        pltpu.sync_copy(dst_hbm.at[wid], idx_v)
        for i in range(CH // 16):
            ones_v[pl.ds(i * 16, 16)] = jnp.ones((16,), jnp.float32)
        plsc.subcore_barrier()
        for g in range(0, E_NCH, 16):
            descs = [
                pltpu.async_copy(ones_v, deg_sh.at[idx_v.at[j]], sem, add=True)
                for j in range(g, g + 16)
            ]
            for d_ in descs:
                d_.wait()
        plsc.subcore_barrier()
        pltpu.sync_copy(deg_sh.at[pl.ds(s * ZR, ZR)], out_hbm.at[c, pl.ds(s * ZR, ZR)])

    return deg_kernel(dst_idx, zeros1)


def _sc_aggregate(table, sd0, sd1):
    @functools.partial(
        pl.kernel,
        out_type=jax.ShapeDtypeStruct((2, NPAD, D), jnp.float32),
        mesh=_mesh(),
        scratch_types=[
            pltpu.VMEM((2, CH), jnp.int32),
            pltpu.VMEM((2, CH), jnp.int32),
            pltpu.VMEM((CH, D), jnp.float32),
            pltpu.VMEM((CH, D), jnp.float32),
            pltpu.VMEM((40, D), jnp.float32),
            pltpu.VMEM_SHARED((NPAD, D), jnp.float32),
            pltpu.SemaphoreType.DMA,
            pltpu.SemaphoreType.DMA,
            pltpu.SemaphoreType.DMA,
            pltpu.SemaphoreType.DMA,
        ],
    )
    def agg_kernel(tab_hbm, sd0_hbm, sd1_hbm, out_hbm,
                   i0, i1, r0, r1, zbuf, acc_sh, sg0, sg1, ss0, ss1):
        c = lax.axis_index("c")
        s = lax.axis_index("s")
        for row in range(40):
            for q in range(D // 16):
                zbuf[row, pl.ds(q * 16, 16)] = jnp.zeros((16,), jnp.float32)
        for k in range(16):
            pltpu.sync_copy(zbuf, acc_sh.at[pl.ds(s * ZR + k * 40, 40)])
        plsc.subcore_barrier()

        ibufs = (i0, i1)
        rbufs = (r0, r1)
        gsems = (sg0, sg1)
        ssems = (ss0, ss1)

        def run(sd_hbm, nch):
            pltpu.sync_copy(sd_hbm.at[s, 0], i0)
            pend_g = pltpu.async_copy(tab_hbm.at[i0.at[0]], r0, sg0)
            pend_s = [None, None]
            for j in range(nch):
                p = j % 2
                pn = (j + 1) % 2
                nxt = None
                if j + 1 < nch:
                    if pend_s[pn] is not None:
                        pend_s[pn].wait()
                        pend_s[pn] = None
                    pltpu.sync_copy(sd_hbm.at[s, j + 1], ibufs[pn])
                    nxt = pltpu.async_copy(tab_hbm.at[ibufs[pn].at[0]], rbufs[pn],
                                           gsems[pn])
                pend_g.wait()
                pend_s[p] = pltpu.async_copy(rbufs[p], acc_sh.at[ibufs[p].at[1]],
                                             ssems[p], add=True)
                pend_g = nxt
            for d_ in pend_s:
                if d_ is not None:
                    d_.wait()

        pl.when(c == 0)(lambda: run(sd0_hbm, E_NCH_C[0]))
        if E_NCH_C[1]:
            pl.when(c == 1)(lambda: run(sd1_hbm, E_NCH_C[1]))
        plsc.subcore_barrier()
        pltpu.sync_copy(acc_sh.at[pl.ds(s * ZR, ZR)], out_hbm.at[c, pl.ds(s * ZR, ZR)])

    return agg_kernel(table, sd0, sd1)


def _sc_pair_gather(table, sidx, didx):
    @functools.partial(
        pl.kernel,
        out_type=(jax.ShapeDtypeStruct((LPAD, D), jnp.float32),
                  jax.ShapeDtypeStruct((LPAD, D), jnp.float32)),
        mesh=_mesh(),
        scratch_types=[
            pltpu.VMEM((L_TNCH, LCH), jnp.int32),
            pltpu.VMEM((L_TNCH, LCH), jnp.int32),
            pltpu.VMEM((LCH, D), jnp.float32),
            pltpu.VMEM((LCH, D), jnp.float32),
            pltpu.VMEM((LCH, D), jnp.float32),
            pltpu.VMEM((LCH, D), jnp.float32),
            pltpu.VMEM_SHARED((NPAD, D), jnp.float32),
            pltpu.SemaphoreType.DMA,
            pltpu.SemaphoreType.DMA,
            pltpu.SemaphoreType.DMA,
            pltpu.SemaphoreType.DMA,
        ],
    )
    def pg_kernel(tab_hbm, s_hbm, d_hbm, outs_hbm, outd_hbm,
                  s_v, d_v, a0, a1, b0, b1, ztab, sa0, sa1, sb0, sb1):
        c = lax.axis_index("c")
        s = lax.axis_index("s")
        wid = c * 16 + s
        base = wid * (L_TNCH * LCH)
        pltpu.sync_copy(tab_hbm.at[pl.ds(s * ZR, ZR)], ztab.at[pl.ds(s * ZR, ZR)])
        pltpu.sync_copy(s_hbm.at[wid], s_v)
        pltpu.sync_copy(d_hbm.at[wid], d_v)
        plsc.subcore_barrier()
        abufs = (a0, a1)
        bbufs = (b0, b1)
        asems = (sa0, sa1)
        bsems = (sb0, sb1)
        pend = (pltpu.async_copy(ztab.at[s_v.at[0]], a0, sa0),
                pltpu.async_copy(ztab.at[d_v.at[0]], b0, sb0))
        for t in range(L_TNCH):
            nxt = None
            if t + 1 < L_TNCH:
                p = (t + 1) % 2
                nxt = (pltpu.async_copy(ztab.at[s_v.at[t + 1]], abufs[p], asems[p]),
                       pltpu.async_copy(ztab.at[d_v.at[t + 1]], bbufs[p], bsems[p]))
            pend[0].wait()
            pend[1].wait()
            pltpu.sync_copy(abufs[t % 2], outs_hbm.at[pl.ds(base + t * LCH, LCH)])
            pltpu.sync_copy(bbufs[t % 2], outd_hbm.at[pl.ds(base + t * LCH, LCH)])
            pend = nxt

    return pg_kernel(table, sidx, didx)


def _tc_encode1(x_pad, degp_t, W1):
    def body(x_ref, dp_ref, w_ref, h_ref, hs_ref, dv_ref):
        dp = dp_ref[...]
        deg = dp[:, 0:1] + dp[:, 1:2] + 1.0
        dinv = lax.rsqrt(jnp.maximum(deg, 1.0))
        dv = jnp.broadcast_to(dinv, (BM, D))
        h = jnp.dot(x_ref[...], w_ref[...],
                    preferred_element_type=jnp.float32,
                    precision=lax.Precision.HIGHEST)
        h_ref[...] = h
        hs_ref[...] = h * dv
        dv_ref[...] = dv

    return pl.pallas_call(
        body,
        grid=(NPAD // BM,),
        in_specs=[pl.BlockSpec((BM, D), lambda i: (i, 0)),
                  pl.BlockSpec((BM, 2), lambda i: (i, 0)),
                  pl.BlockSpec((D, D), lambda i: (0, 0))],
        out_specs=[pl.BlockSpec((BM, D), lambda i: (i, 0))] * 3,
        out_shape=[jax.ShapeDtypeStruct((NPAD, D), jnp.float32)] * 3,
    )(x_pad, degp_t, W1)


def _tc_mid(acc0, acc1, h1, dv2, b1, W2):
    def body(a0_ref, a1_ref, h_ref, dv_ref, b_ref, w_ref, h2_ref, hs2_ref):
        dv = dv_ref[...]
        z1 = dv * (a0_ref[...] + a1_ref[...] + dv * h_ref[...]) + b_ref[...]
        z1 = jnp.maximum(z1, 0.0)
        h2 = jnp.dot(z1, w_ref[...],
                     preferred_element_type=jnp.float32,
                     precision=lax.Precision.HIGHEST)
        i = pl.program_id(0)
        rid = i * BM + lax.broadcasted_iota(jnp.int32, (BM, D), 0)
        h2_ref[...] = h2
        hs2_ref[...] = jnp.where(rid < N, h2 * dv, 0.0)

    return pl.pallas_call(
        body,
        grid=(NPAD // BM,),
        in_specs=[pl.BlockSpec((BM, D), lambda i: (i, 0)),
                  pl.BlockSpec((BM, D), lambda i: (i, 0)),
                  pl.BlockSpec((BM, D), lambda i: (i, 0)),
                  pl.BlockSpec((BM, D), lambda i: (i, 0)),
                  pl.BlockSpec((1, D), lambda i: (0, 0)),
                  pl.BlockSpec((D, D), lambda i: (0, 0))],
        out_specs=[pl.BlockSpec((BM, D), lambda i: (i, 0))] * 2,
        out_shape=[jax.ShapeDtypeStruct((NPAD, D), jnp.float32)] * 2,
    )(acc0, acc1, h1, dv2, b1, W2)


def _tc_final(acc0, acc1, h2, dv2, b2):
    def body(a0_ref, a1_ref, h_ref, dv_ref, b_ref, z_ref):
        dv = dv_ref[...]
        z_ref[...] = dv * (a0_ref[...] + a1_ref[...] + dv * h_ref[...]) + b_ref[...]

    return pl.pallas_call(
        body,
        grid=(NPAD // BM,),
        in_specs=[pl.BlockSpec((BM, D), lambda i: (i, 0)),
                  pl.BlockSpec((BM, D), lambda i: (i, 0)),
                  pl.BlockSpec((BM, D), lambda i: (i, 0)),
                  pl.BlockSpec((BM, D), lambda i: (i, 0)),
                  pl.BlockSpec((1, D), lambda i: (0, 0))],
        out_specs=pl.BlockSpec((BM, D), lambda i: (i, 0)),
        out_shape=jax.ShapeDtypeStruct((NPAD, D), jnp.float32),
    )(acc0, acc1, h2, dv2, b2)


def _tc_dot(zs, zd):
    BL = 2048

    def body(a_ref, b_ref, o_ref):
        o_ref[...] = jnp.sum(a_ref[...] * b_ref[...], axis=1, keepdims=True)

    return pl.pallas_call(
        body,
        grid=(LPAD // BL,),
        in_specs=[pl.BlockSpec((BL, D), lambda i: (i, 0)),
                  pl.BlockSpec((BL, D), lambda i: (i, 0))],
        out_specs=pl.BlockSpec((BL, 1), lambda i: (i, 0)),
        out_shape=jax.ShapeDtypeStruct((LPAD, 1), jnp.float32),
    )(zs, zd)


def kernel(x, edge_index, edge_label_index, W1, b1, W2, b2):
    f32 = jnp.float32
    src = edge_index[0]
    dst = edge_index[1]
    pad_e = jnp.full((EPAD - E,), N, jnp.int32)
    srcf = jnp.concatenate([src, pad_e])
    dstf = jnp.concatenate([dst, pad_e])
    dstp = dstf.reshape(NW, E_NCH, CH)
    ne0 = 16 * E_NCH_C[0] * CH
    sd0 = jnp.stack([srcf[:ne0].reshape(16, E_NCH_C[0], CH),
                     dstf[:ne0].reshape(16, E_NCH_C[0], CH)], axis=2)
    sd1 = jnp.stack([srcf[ne0:].reshape(16, E_NCH_C[1], CH),
                     dstf[ne0:].reshape(16, E_NCH_C[1], CH)], axis=2)
    pad_l = jnp.zeros((LPAD - L,), jnp.int32)
    s2p = jnp.concatenate([edge_label_index[0], pad_l]).reshape(NW, L_TNCH, LCH)
    d2p = jnp.concatenate([edge_label_index[1], pad_l]).reshape(NW, L_TNCH, LCH)
    zeros1 = jnp.zeros((NPAD,), f32)
    x_pad = jnp.concatenate([x, jnp.zeros((NPAD - N, D), f32)])

    degp = _sc_degree(dstp, zeros1)
    h1, hs1, dv2 = _tc_encode1(x_pad, degp.T, W1)
    accp1 = _sc_aggregate(hs1, sd0, sd1)
    h2, hs2 = _tc_mid(accp1[0], accp1[1], h1, dv2, b1.reshape(1, D), W2)
    accp2 = _sc_aggregate(hs2, sd0, sd1)
    z2 = _tc_final(accp2[0], accp2[1], h2, dv2, b2.reshape(1, D))
    zs, zd = _sc_pair_gather(z2, s2p, d2p)
    scores = _tc_dot(zs, zd)
    return scores.reshape(LPAD)[:L]


# speedup vs baseline: 1.2304x; 1.2304x over previous
import functools

import jax
import jax.numpy as jnp
from jax import lax
from jax.experimental import pallas as pl
from jax.experimental.pallas import tpu as pltpu
from jax.experimental.pallas import tpu_sc as plsc

N = 10000
D = 128
NPAD = 10240
ZR = NPAD // 16
E = 320000
NW = 32
CH = 128
E_NCH = 80
EPAD = NW * E_NCH * CH
L = 100000
L_NCH = 25
LPAD = NW * L_NCH * CH
BM = 1024

E_NCH_C = (150, 10)
LCH = 64
L_TNCH = 50


def _mesh():
    return plsc.VectorSubcoreMesh(core_axis_name="c", subcore_axis_name="s")


def _sc_degree(dst_idx, zeros1):
    @functools.partial(
        pl.kernel,
        out_type=jax.ShapeDtypeStruct((2, NPAD), jnp.float32),
        mesh=_mesh(),
        scratch_types=[
            pltpu.VMEM((E_NCH, CH), jnp.int32),
            pltpu.VMEM((CH,), jnp.float32),
            pltpu.VMEM_SHARED((NPAD,), jnp.float32),
            pltpu.SemaphoreType.DMA,
        ],
    )
    def deg_kernel(dst_hbm, z_hbm, out_hbm, idx_v, ones_v, deg_sh, sem):
        c = lax.axis_index("c")
        s = lax.axis_index("s")
        wid = c * 16 + s
        pltpu.sync_copy(z_hbm.at[pl.ds(s * ZR, ZR)], deg_sh.at[pl.ds(s * ZR, ZR)])
        pltpu.sync_copy(dst_hbm.at[wid], idx_v)
        for i in range(CH // 16):
            ones_v[pl.ds(i * 16, 16)] = jnp.ones((16,), jnp.float32)
        plsc.subcore_barrier()
        for g in range(0, E_NCH, 16):
            descs = [
                pltpu.async_copy(ones_v, deg_sh.at[idx_v.at[j]], sem, add=True)
                for j in range(g, g + 16)
            ]
            for d_ in descs:
                d_.wait()
        plsc.subcore_barrier()
        pltpu.sync_copy(deg_sh.at[pl.ds(s * ZR, ZR)], out_hbm.at[c, pl.ds(s * ZR, ZR)])

    return deg_kernel(dst_idx, zeros1)


def _sc_aggregate(table, sd0, sd1):
    @functools.partial(
        pl.kernel,
        out_type=jax.ShapeDtypeStruct((2, NPAD, D), jnp.float32),
        mesh=_mesh(),
        scratch_types=[
            pltpu.VMEM((2, CH), jnp.int32),
            pltpu.VMEM((2, CH), jnp.int32),
            pltpu.VMEM((CH, D), jnp.float32),
            pltpu.VMEM((CH, D), jnp.float32),
            pltpu.VMEM((40, D), jnp.float32),
            pltpu.VMEM_SHARED((NPAD, D), jnp.float32),
            pltpu.SemaphoreType.DMA,
            pltpu.SemaphoreType.DMA,
            pltpu.SemaphoreType.DMA,
            pltpu.SemaphoreType.DMA,
        ],
    )
    def agg_kernel(tab_hbm, sd0_hbm, sd1_hbm, out_hbm,
                   i0, i1, r0, r1, zbuf, acc_sh, sg0, sg1, ss0, ss1):
        c = lax.axis_index("c")
        s = lax.axis_index("s")
        for row in range(40):
            for q in range(D // 16):
                zbuf[row, pl.ds(q * 16, 16)] = jnp.zeros((16,), jnp.float32)
        for k in range(16):
            pltpu.sync_copy(zbuf, acc_sh.at[pl.ds(s * ZR + k * 40, 40)])
        plsc.subcore_barrier()

        ibufs = (i0, i1)
        rbufs = (r0, r1)
        gsems = (sg0, sg1)
        ssems = (ss0, ss1)

        def run(sd_hbm, nch):
            pltpu.sync_copy(sd_hbm.at[s, 0], i0)
            pend_g = pltpu.async_copy(tab_hbm.at[i0.at[0]], r0, sg0)
            pend_s = [None, None]
            for j in range(nch):
                p = j % 2
                pn = (j + 1) % 2
                nxt = None
                if j + 1 < nch:
                    if pend_s[pn] is not None:
                        pend_s[pn].wait()
                        pend_s[pn] = None
                    pltpu.sync_copy(sd_hbm.at[s, j + 1], ibufs[pn])
                    nxt = pltpu.async_copy(tab_hbm.at[ibufs[pn].at[0]], rbufs[pn],
                                           gsems[pn])
                pend_g.wait()
                pend_s[p] = pltpu.async_copy(rbufs[p], acc_sh.at[ibufs[p].at[1]],
                                             ssems[p], add=True)
                pend_g = nxt
            for d_ in pend_s:
                if d_ is not None:
                    d_.wait()

        pl.when(c == 0)(lambda: run(sd0_hbm, E_NCH_C[0]))
        if E_NCH_C[1]:
            pl.when(c == 1)(lambda: run(sd1_hbm, E_NCH_C[1]))
        plsc.subcore_barrier()
        pltpu.sync_copy(acc_sh.at[pl.ds(s * ZR, ZR)], out_hbm.at[c, pl.ds(s * ZR, ZR)])

    return agg_kernel(table, sd0, sd1)


def _sc_pair_gather(table, sidx, didx):
    @functools.partial(
        pl.kernel,
        out_type=(jax.ShapeDtypeStruct((LPAD, D), jnp.float32),
                  jax.ShapeDtypeStruct((LPAD, D), jnp.float32)),
        mesh=_mesh(),
        scratch_types=[
            pltpu.VMEM((L_TNCH, LCH), jnp.int32),
            pltpu.VMEM((L_TNCH, LCH), jnp.int32),
            pltpu.VMEM((LCH, D), jnp.float32),
            pltpu.VMEM((LCH, D), jnp.float32),
            pltpu.VMEM((LCH, D), jnp.float32),
            pltpu.VMEM((LCH, D), jnp.float32),
            pltpu.VMEM_SHARED((NPAD, D), jnp.float32),
            pltpu.SemaphoreType.DMA,
            pltpu.SemaphoreType.DMA,
            pltpu.SemaphoreType.DMA,
            pltpu.SemaphoreType.DMA,
        ],
    )
    def pg_kernel(tab_hbm, s_hbm, d_hbm, outs_hbm, outd_hbm,
                  s_v, d_v, a0, a1, b0, b1, ztab, sa0, sa1, sb0, sb1):
        c = lax.axis_index("c")
        s = lax.axis_index("s")
        wid = c * 16 + s
        base = wid * (L_TNCH * LCH)
        pltpu.sync_copy(tab_hbm.at[pl.ds(s * ZR, ZR)], ztab.at[pl.ds(s * ZR, ZR)])
        pltpu.sync_copy(s_hbm.at[wid], s_v)
        pltpu.sync_copy(d_hbm.at[wid], d_v)
        plsc.subcore_barrier()
        abufs = (a0, a1)
        bbufs = (b0, b1)
        asems = (sa0, sa1)
        bsems = (sb0, sb1)
        pend = (pltpu.async_copy(ztab.at[s_v.at[0]], a0, sa0),
                pltpu.async_copy(ztab.at[d_v.at[0]], b0, sb0))
        for t in range(L_TNCH):
            nxt = None
            if t + 1 < L_TNCH:
                p = (t + 1) % 2
                nxt = (pltpu.async_copy(ztab.at[s_v.at[t + 1]], abufs[p], asems[p]),
                       pltpu.async_copy(ztab.at[d_v.at[t + 1]], bbufs[p], bsems[p]))
            pend[0].wait()
            pend[1].wait()
            pltpu.sync_copy(abufs[t % 2], outs_hbm.at[pl.ds(base + t * LCH, LCH)])
            pltpu.sync_copy(bbufs[t % 2], outd_hbm.at[pl.ds(base + t * LCH, LCH)])
            pend = nxt

    return pg_kernel(table, sidx, didx)


def _tc_encode1(x_pad, degp_t, W1):
    def body(x_ref, dp_ref, w_ref, h_ref, hs_ref, dv_ref):
        dp = dp_ref[...]
        deg = dp[:, 0:1] + dp[:, 1:2] + 1.0
        dinv = lax.rsqrt(jnp.maximum(deg, 1.0))
        dv = jnp.broadcast_to(dinv, (BM, D))
        h = jnp.dot(x_ref[...], w_ref[...],
                    preferred_element_type=jnp.float32,
                    precision=lax.Precision.HIGHEST)
        h_ref[...] = h
        hs_ref[...] = h * dv
        dv_ref[...] = dv

    return pl.pallas_call(
        body,
        grid=(NPAD // BM,),
        in_specs=[pl.BlockSpec((BM, D), lambda i: (i, 0)),
                  pl.BlockSpec((BM, 2), lambda i: (i, 0)),
                  pl.BlockSpec((D, D), lambda i: (0, 0))],
        out_specs=[pl.BlockSpec((BM, D), lambda i: (i, 0))] * 3,
        out_shape=[jax.ShapeDtypeStruct((NPAD, D), jnp.float32)] * 3,
    )(x_pad, degp_t, W1)


def _tc_mid(acc0, acc1, h1, dv2, b1, W2):
    def body(a0_ref, a1_ref, h_ref, dv_ref, b_ref, w_ref, h2_ref, hs2_ref):
        dv = dv_ref[...]
        z1 = dv * (a0_ref[...] + a1_ref[...] + dv * h_ref[...]) + b_ref[...]
        z1 = jnp.maximum(z1, 0.0)
        h2 = jnp.dot(z1, w_ref[...],
                     preferred_element_type=jnp.float32,
                     precision=lax.Precision.HIGHEST)
        i = pl.program_id(0)
        rid = i * BM + lax.broadcasted_iota(jnp.int32, (BM, D), 0)
        h2_ref[...] = h2
        hs2_ref[...] = jnp.where(rid < N, h2 * dv, 0.0)

    return pl.pallas_call(
        body,
        grid=(NPAD // BM,),
        in_specs=[pl.BlockSpec((BM, D), lambda i: (i, 0)),
                  pl.BlockSpec((BM, D), lambda i: (i, 0)),
                  pl.BlockSpec((BM, D), lambda i: (i, 0)),
                  pl.BlockSpec((BM, D), lambda i: (i, 0)),
                  pl.BlockSpec((1, D), lambda i: (0, 0)),
                  pl.BlockSpec((D, D), lambda i: (0, 0))],
        out_specs=[pl.BlockSpec((BM, D), lambda i: (i, 0))] * 2,
        out_shape=[jax.ShapeDtypeStruct((NPAD, D), jnp.float32)] * 2,
    )(acc0, acc1, h1, dv2, b1, W2)


def _tc_final(acc0, acc1, h2, dv2, b2):
    def body(a0_ref, a1_ref, h_ref, dv_ref, b_ref, z_ref):
        dv = dv_ref[...]
        z_ref[...] = dv * (a0_ref[...] + a1_ref[...] + dv * h_ref[...]) + b_ref[...]

    return pl.pallas_call(
        body,
        grid=(NPAD // BM,),
        in_specs=[pl.BlockSpec((BM, D), lambda i: (i, 0)),
                  pl.BlockSpec((BM, D), lambda i: (i, 0)),
                  pl.BlockSpec((BM, D), lambda i: (i, 0)),
                  pl.BlockSpec((BM, D), lambda i: (i, 0)),
                  pl.BlockSpec((1, D), lambda i: (0, 0))],
        out_specs=pl.BlockSpec((BM, D), lambda i: (i, 0)),
        out_shape=jax.ShapeDtypeStruct((NPAD, D), jnp.float32),
    )(acc0, acc1, h2, dv2, b2)


def _tc_dot(zs, zd):
    BL = 2048

    def body(a_ref, b_ref, o_ref):
        o_ref[...] = jnp.sum(a_ref[...] * b_ref[...], axis=1, keepdims=True)

    return pl.pallas_call(
        body,
        grid=(LPAD // BL,),
        in_specs=[pl.BlockSpec((BL, D), lambda i: (i, 0)),
                  pl.BlockSpec((BL, D), lambda i: (i, 0))],
        out_specs=pl.BlockSpec((BL, 1), lambda i: (i, 0)),
        out_shape=jax.ShapeDtypeStruct((LPAD, 1), jnp.float32),
    )(zs, zd)


def kernel(x, edge_index, edge_label_index, W1, b1, W2, b2):
    f32 = jnp.float32
    src = edge_index[0]
    dst = edge_index[1]
    pad_e = jnp.full((EPAD - E,), N, jnp.int32)
    srcf = jnp.concatenate([src, pad_e])
    dstf = jnp.concatenate([dst, pad_e])
    dstp = dstf.reshape(NW, E_NCH, CH)
    ne0 = 16 * E_NCH_C[0] * CH
    sd0 = jnp.stack([srcf[:ne0].reshape(16, E_NCH_C[0], CH),
                     dstf[:ne0].reshape(16, E_NCH_C[0], CH)], axis=2)
    sd1 = jnp.stack([srcf[ne0:].reshape(16, E_NCH_C[1], CH),
                     dstf[ne0:].reshape(16, E_NCH_C[1], CH)], axis=2)
    pad_l = jnp.zeros((LPAD - L,), jnp.int32)
    s2p = jnp.concatenate([edge_label_index[0], pad_l]).reshape(NW, L_TNCH, LCH)
    d2p = jnp.concatenate([edge_label_index[1], pad_l]).reshape(NW, L_TNCH, LCH)
    zeros1 = jnp.zeros((NPAD,), f32)
    x_pad = jnp.concatenate([x, jnp.zeros((NPAD - N, D), f32)])

    degp = _sc_degree(dstp, zeros1)
    h1, hs1, dv2 = _tc_encode1(x_pad, degp.T, W1)
    accp1 = _sc_aggregate(hs1, sd0, sd1)
    h2, hs2 = _tc_mid(accp1[0], accp1[1], h1, dv2, b1.reshape(1, D), W2)
    accp2 = _sc_aggregate(hs2, sd0, sd1)
    z2 = _tc_final(accp2[0], accp2[1], h2, dv2, b2.reshape(1, D))
    zs, zd = _sc_pair_gather(z2, s2p, d2p)
    scores = _tc_dot(zs, zd)
    return scores.reshape(LPAD)[:L]
